# baseline (speedup 1.0000x reference)
import functools

import jax
import jax.numpy as jnp
from jax import lax
from jax.experimental import pallas as pl
from jax.experimental.pallas import tpu as pltpu
from jax.experimental.pallas import tpu_sc as plsc

_V = 1000000
_D = 32
_B = 16384 * 50
_NC, _NS = 2, 16
_NW = _NC * _NS
_ROWS_PER_W = _B // _NW
_GRP = 128
_NGRP = _ROWS_PER_W // _GRP
_CG = 4
_NCHUNK = _NGRP // _CG


_NBLK = 1000000 // 128
_BLK_PER_W = _NBLK // _NW


def _make_transpose():
  mesh = plsc.VectorSubcoreMesh(core_axis_name="c", subcore_axis_name="s")

  @functools.partial(
      pl.kernel,
      out_type=jax.ShapeDtypeStruct((_V // 4, 128), jnp.float32),
      mesh=mesh,
      scratch_types=[
          pltpu.VMEM((_D, _GRP), jnp.float32),
          pltpu.VMEM((_D, _GRP), jnp.float32),
          pltpu.VMEM((_D, _GRP), jnp.float32),
          pltpu.VMEM((_D, _GRP), jnp.float32),
          pltpu.SemaphoreType.DMA,
          pltpu.SemaphoreType.DMA,
          pltpu.SemaphoreType.DMA,
          pltpu.SemaphoreType.DMA,
      ],
      compiler_params=pltpu.CompilerParams(
          use_tc_tiling_on_sc=True, needs_layout_passes=False),
  )
  def k(embt_hbm, tail_hbm, out_hbm, in_a, in_b, out_a, out_b,
        isem_a, isem_b, wsem_a, wsem_b):
    wid = lax.axis_index("s") * _NC + lax.axis_index("c")
    iota16 = lax.iota(jnp.int32, 16)

    def fire_in(c, buf, sem):
      pltpu.async_copy(embt_hbm.at[:, pl.ds(c * _GRP, _GRP)], buf, sem)

    def drain_in(sem, buf):
      pltpu.make_async_copy(embt_hbm.at[:, pl.ds(0, _GRP)], buf, sem).wait()

    def fire_out(c, buf, sem):
      pltpu.async_copy(buf.at[pl.ds(0, _D)], out_hbm.at[pl.ds(c * 32, 32)],
                       sem)

    def drain_out(sem, buf):
      pltpu.make_async_copy(out_hbm.at[pl.ds(0, 32)], buf.at[pl.ds(0, _D)],
                            sem).wait()

    def transpose_blk(src, dst, nrow=_D):
      for m in range(8):
        row = iota16 + 16 * (m & 1)
        for i in range(nrow):
          col = jnp.full((16,), 4 * i + (m >> 1), jnp.int32)
          dst[i, pl.ds(16 * m, 16)] = plsc.load_gather(src, [row, col])

    fire_in(wid, in_a, isem_a)

    @pl.loop(0, _BLK_PER_W, step=2)
    def _blocks(j0):
      c0 = wid + j0 * _NW
      drain_in(isem_a, in_a)
      fire_in(c0 + _NW, in_b, isem_b)
      @pl.when(j0 >= 2)
      def _():
        drain_out(wsem_a, out_a)
      transpose_blk(in_a, out_a)
      fire_out(c0, out_a, wsem_a)
      drain_in(isem_b, in_b)
      @pl.when(j0 + 2 < _BLK_PER_W)
      def _():
        fire_in(c0 + 2 * _NW, in_a, isem_a)
      @pl.when(j0 >= 1)
      def _():
        drain_out(wsem_b, out_b)
      transpose_blk(in_b, out_b)
      fire_out(c0 + _NW, out_b, wsem_b)

    drain_out(wsem_a, out_a)
    drain_out(wsem_b, out_b)

    @pl.when(wid < 4)
    def _():
      c = _BLK_PER_W * _NW + wid
      pltpu.sync_copy(embt_hbm.at[:, pl.ds(c * _GRP, _GRP)], in_a)
      transpose_blk(in_a, out_a)
      pltpu.sync_copy(out_a.at[pl.ds(0, _D)], out_hbm.at[pl.ds(c * 32, 32)])

    @pl.when(wid == 4)
    def _():
      pltpu.sync_copy(tail_hbm, in_a.at[pl.ds(0, 16)])
      pltpu.sync_copy(in_a.at[pl.ds(0, 16)],
                      out_hbm.at[pl.ds(_NBLK * 32, 16)])

  return k


def _make_gather():
  mesh = plsc.VectorSubcoreMesh(core_axis_name="c", subcore_axis_name="s")

  @functools.partial(
      pl.kernel,
      out_type=jax.ShapeDtypeStruct((4, _B // _GRP, 8, _GRP), jnp.float32),
      mesh=mesh,
      scratch_types=[
          pltpu.VMEM((_NGRP, _GRP), jnp.int32),
          pltpu.VMEM((_CG * _GRP, _D), jnp.float32),
          pltpu.VMEM((_CG * _GRP, _D), jnp.float32),
          pltpu.VMEM((4, _CG, 8, _GRP), jnp.float32),
          pltpu.VMEM((4, _CG, 8, _GRP), jnp.float32),
          pltpu.SemaphoreType.DMA,
          pltpu.SemaphoreType.DMA,
          pltpu.SemaphoreType.DMA,
          pltpu.SemaphoreType.DMA,
      ],
      compiler_params=pltpu.CompilerParams(
          use_tc_tiling_on_sc=False, needs_layout_passes=False),
  )
  def k(src_hbm, tab_hbm, out_hbm, idx_v, rows_a, rows_b, oblk_a, oblk_b,
        gsem_a, gsem_b, osem_a, osem_b):
    wid = lax.axis_index("s") * _NC + lax.axis_index("c")
    pltpu.sync_copy(src_hbm.at[wid], idx_v)
    gbase = wid * _NGRP
    iota16 = lax.iota(jnp.int32, 16)

    def fire_gathers(c, rows, sem):
      for g in range(_CG):
        pltpu.async_copy(
            tab_hbm.at[idx_v.at[c * _CG + g]],
            rows.at[pl.ds(g * _GRP, _GRP)], sem)

    def drain_g(sem, rows):
      pltpu.make_async_copy(tab_hbm.at[pl.ds(0, _CG * _GRP)], rows, sem).wait()

    def drain_o(sem, oblk):
      pltpu.make_async_copy(out_hbm.at[:, pl.ds(0, _CG)], oblk, sem).wait()

    def transpose_chunk(rows, oblk):
      @pl.loop(0, _CG)
      def _g(g):
        for m in range(8):
          row = iota16 + (g * _GRP + 16 * m)
          for r in range(4):
            for s in range(8):
              col = jnp.full((16,), 8 * r + s, jnp.int32)
              oblk[r, g, s, pl.ds(16 * m, 16)] = plsc.load_gather(
                  rows, [row, col])

    def fire_out(c, oblk, sem):
      pltpu.async_copy(
          oblk, out_hbm.at[:, pl.ds(gbase + c * _CG, _CG)], sem)

    fire_gathers(0, rows_a, gsem_a)

    @pl.loop(0, _NCHUNK, step=2)
    def _chunks(c0):
      drain_g(gsem_a, rows_a)
      fire_gathers(c0 + 1, rows_b, gsem_b)
      @pl.when(c0 >= 2)
      def _():
        drain_o(osem_a, oblk_a)
      transpose_chunk(rows_a, oblk_a)
      fire_out(c0, oblk_a, osem_a)
      drain_g(gsem_b, rows_b)
      @pl.when(c0 + 2 < _NCHUNK)
      def _():
        fire_gathers(c0 + 2, rows_a, gsem_a)
      @pl.when(c0 >= 1)
      def _():
        drain_o(osem_b, oblk_b)
      transpose_chunk(rows_b, oblk_b)
      fire_out(c0 + 1, oblk_b, osem_b)

    drain_o(osem_a, oblk_a)
    drain_o(osem_b, oblk_b)

  return k


_transpose = _make_transpose()
_gather = _make_gather()


def kernel(inputs, embeddings):
  embt = jnp.transpose(embeddings)
  tail_rm = jnp.reshape(embeddings[_NBLK * _GRP:, :], (16, 128))
  table_pk = _transpose(embt, tail_rm)
  tab = jnp.reshape(table_pk, (_V, _D))
  src = jnp.reshape(inputs.astype(jnp.int32), (_NW, _NGRP, _GRP))
  out4 = _gather(src, tab)
  return jnp.reshape(jnp.transpose(out4, (1, 3, 0, 2)), (_B, _D))


# speedup vs baseline: 2.0204x; 2.0204x over previous
import functools

import jax
import jax.numpy as jnp
from jax import lax
from jax.experimental import pallas as pl
from jax.experimental.pallas import tpu as pltpu
from jax.experimental.pallas import tpu_sc as plsc

_V = 1000000
_D = 32
_B = 16384 * 50
_NC, _NS = 2, 16
_NW = _NC * _NS
_ROWS_PER_W = _B // _NW
_GRP = 128
_NGRP = _ROWS_PER_W // _GRP
_CG = 4
_NCHUNK = _NGRP // _CG


_NBLK = 1000000 // 128
_BLK_PER_W = _NBLK // _NW


def _make_transpose():
  mesh = plsc.VectorSubcoreMesh(core_axis_name="c", subcore_axis_name="s")

  @functools.partial(
      pl.kernel,
      out_type=jax.ShapeDtypeStruct((_V // 4, 128), jnp.float32),
      mesh=mesh,
      scratch_types=[
          pltpu.VMEM((_D, _GRP), jnp.float32),
          pltpu.VMEM((_D, _GRP), jnp.float32),
          pltpu.VMEM((_D, _GRP), jnp.float32),
          pltpu.VMEM((_D, _GRP), jnp.float32),
          pltpu.SemaphoreType.DMA,
          pltpu.SemaphoreType.DMA,
          pltpu.SemaphoreType.DMA,
          pltpu.SemaphoreType.DMA,
      ],
      compiler_params=pltpu.CompilerParams(
          use_tc_tiling_on_sc=True, needs_layout_passes=False),
  )
  def k(embt_hbm, tail_hbm, out_hbm, in_a, in_b, out_a, out_b,
        isem_a, isem_b, wsem_a, wsem_b):
    wid = lax.axis_index("s") * _NC + lax.axis_index("c")
    iota16 = lax.iota(jnp.int32, 16)

    def fire_in(c, buf, sem):
      pltpu.async_copy(embt_hbm.at[:, pl.ds(c * _GRP, _GRP)], buf, sem)

    def drain_in(sem, buf):
      pltpu.make_async_copy(embt_hbm.at[:, pl.ds(0, _GRP)], buf, sem).wait()

    def fire_out(c, buf, sem):
      pltpu.async_copy(buf.at[pl.ds(0, _D)], out_hbm.at[pl.ds(c * 32, 32)],
                       sem)

    def drain_out(sem, buf):
      pltpu.make_async_copy(out_hbm.at[pl.ds(0, 32)], buf.at[pl.ds(0, _D)],
                            sem).wait()

    def transpose_blk(src, dst):
      for m in range(8):
        a_vec = iota16 + 16 * m
        f_base = a_vec * 32
        for k in range(32):
          b_vec = (iota16 + k) & 31
          v = plsc.load_gather(src, [b_vec, a_vec])
          f = f_base + b_vec
          plsc.store_scatter(
              dst, [lax.shift_right_logical(f, 7), f & 127], v)

    fire_in(wid, in_a, isem_a)

    @pl.loop(0, _BLK_PER_W, step=2)
    def _blocks(j0):
      c0 = wid + j0 * _NW
      drain_in(isem_a, in_a)
      fire_in(c0 + _NW, in_b, isem_b)
      @pl.when(j0 >= 2)
      def _():
        drain_out(wsem_a, out_a)
      transpose_blk(in_a, out_a)
      fire_out(c0, out_a, wsem_a)
      drain_in(isem_b, in_b)
      @pl.when(j0 + 2 < _BLK_PER_W)
      def _():
        fire_in(c0 + 2 * _NW, in_a, isem_a)
      @pl.when(j0 >= 1)
      def _():
        drain_out(wsem_b, out_b)
      transpose_blk(in_b, out_b)
      fire_out(c0 + _NW, out_b, wsem_b)

    drain_out(wsem_a, out_a)
    drain_out(wsem_b, out_b)

    @pl.when(wid < 4)
    def _():
      c = _BLK_PER_W * _NW + wid
      pltpu.sync_copy(embt_hbm.at[:, pl.ds(c * _GRP, _GRP)], in_a)
      transpose_blk(in_a, out_a)
      pltpu.sync_copy(out_a.at[pl.ds(0, _D)], out_hbm.at[pl.ds(c * 32, 32)])

    @pl.when(wid == 4)
    def _():
      pltpu.sync_copy(tail_hbm, in_a.at[pl.ds(0, 16)])
      pltpu.sync_copy(in_a.at[pl.ds(0, 16)],
                      out_hbm.at[pl.ds(_NBLK * 32, 16)])

  return k


def _make_gather():
  mesh = plsc.VectorSubcoreMesh(core_axis_name="c", subcore_axis_name="s")

  @functools.partial(
      pl.kernel,
      out_type=jax.ShapeDtypeStruct((4, _B // _GRP, 8, _GRP), jnp.float32),
      mesh=mesh,
      scratch_types=[
          pltpu.VMEM((_NGRP, _GRP), jnp.int32),
          pltpu.VMEM((_CG * _GRP, _D), jnp.float32),
          pltpu.VMEM((_CG * _GRP, _D), jnp.float32),
          pltpu.VMEM((_D, _CG * _GRP), jnp.float32),
          pltpu.VMEM((_D, _CG * _GRP), jnp.float32),
          pltpu.SemaphoreType.DMA,
          pltpu.SemaphoreType.DMA,
          pltpu.SemaphoreType.DMA,
          pltpu.SemaphoreType.DMA,
      ],
      compiler_params=pltpu.CompilerParams(
          use_tc_tiling_on_sc=False, needs_layout_passes=False),
  )
  def k(src_hbm, tab_hbm, out_hbm, idx_v, rows_a, rows_b, oblk_a, oblk_b,
        gsem_a, gsem_b, osem_a, osem_b):
    wid = lax.axis_index("s") * _NC + lax.axis_index("c")
    pltpu.sync_copy(src_hbm.at[wid], idx_v)
    gbase = wid * _NGRP
    iota16 = lax.iota(jnp.int32, 16)

    def fire_gathers(c, rows, sem):
      for g in range(_CG):
        pltpu.async_copy(
            tab_hbm.at[idx_v.at[c * _CG + g]],
            rows.at[pl.ds(g * _GRP, _GRP)], sem)

    def drain_g(sem, rows):
      pltpu.make_async_copy(tab_hbm.at[pl.ds(0, _CG * _GRP)], rows, sem).wait()

    def drain_o(sem, oblk):
      for r in range(4):
        for g in range(_CG):
          pltpu.make_async_copy(
              out_hbm.at[0, 0],
              oblk.at[pl.ds(8 * r, 8), pl.ds(g * _GRP, _GRP)], sem).wait()

    def transpose_chunk(rows, oblk):
      @pl.loop(0, _CG)
      def _g(g):
        for m in range(8):
          col_vec = iota16 + (g * _GRP + 16 * m)
          for k in range(32):
            d_vec = (iota16 + k) & 31
            v = plsc.load_gather(rows, [col_vec, d_vec])
            plsc.store_scatter(oblk, [d_vec, col_vec], v)

    def fire_out(c, oblk, sem):
      for r in range(4):
        for g in range(_CG):
          pltpu.async_copy(
              oblk.at[pl.ds(8 * r, 8), pl.ds(g * _GRP, _GRP)],
              out_hbm.at[r, gbase + c * _CG + g], sem)

    fire_gathers(0, rows_a, gsem_a)

    @pl.loop(0, _NCHUNK, step=2)
    def _chunks(c0):
      drain_g(gsem_a, rows_a)
      fire_gathers(c0 + 1, rows_b, gsem_b)
      @pl.when(c0 >= 2)
      def _():
        drain_o(osem_a, oblk_a)
      transpose_chunk(rows_a, oblk_a)
      fire_out(c0, oblk_a, osem_a)
      drain_g(gsem_b, rows_b)
      @pl.when(c0 + 2 < _NCHUNK)
      def _():
        fire_gathers(c0 + 2, rows_a, gsem_a)
      @pl.when(c0 >= 1)
      def _():
        drain_o(osem_b, oblk_b)
      transpose_chunk(rows_b, oblk_b)
      fire_out(c0 + 1, oblk_b, osem_b)

    drain_o(osem_a, oblk_a)
    drain_o(osem_b, oblk_b)

  return k


_transpose = _make_transpose()
_gather = _make_gather()


def kernel(inputs, embeddings):
  embt = jnp.transpose(embeddings)
  tail_rm = jnp.reshape(embeddings[_NBLK * _GRP:, :], (16, 128))
  table_pk = _transpose(embt, tail_rm)
  tab = jnp.reshape(table_pk, (_V, _D))
  src = jnp.reshape(inputs.astype(jnp.int32), (_NW, _NGRP, _GRP))
  out4 = _gather(src, tab)
  return jnp.reshape(jnp.transpose(out4, (1, 3, 0, 2)), (_B, _D))


# speedup vs baseline: 4.3351x; 2.1456x over previous
import functools

import jax
import jax.numpy as jnp
from jax import lax
from jax.experimental import pallas as pl
from jax.experimental.pallas import tpu as pltpu
from jax.experimental.pallas import tpu_sc as plsc

_V = 1000000
_D = 32
_B = 16384 * 50
_NC, _NS = 2, 16
_NW = _NC * _NS
_ROWS_PER_W = _B // _NW
_GRP = 128
_NGRP = _ROWS_PER_W // _GRP
_CG = 4
_NCHUNK = _NGRP // _CG


_NBLK = 1000000 // 128
_BLK_PER_W = _NBLK // _NW


def _make_transpose():
  mesh = plsc.VectorSubcoreMesh(core_axis_name="c", subcore_axis_name="s")

  @functools.partial(
      pl.kernel,
      out_type=jax.ShapeDtypeStruct((_V // 4, 128), jnp.float32),
      mesh=mesh,
      scratch_types=[
          pltpu.VMEM((_D, _GRP), jnp.float32),
          pltpu.VMEM((_D, _GRP), jnp.float32),
          pltpu.VMEM((_D, _GRP), jnp.float32),
          pltpu.VMEM((_D, _GRP), jnp.float32),
          pltpu.SemaphoreType.DMA,
          pltpu.SemaphoreType.DMA,
          pltpu.SemaphoreType.DMA,
          pltpu.SemaphoreType.DMA,
      ],
      compiler_params=pltpu.CompilerParams(
          use_tc_tiling_on_sc=True, needs_layout_passes=False),
  )
  def k(embt_hbm, tail_hbm, out_hbm, in_a, in_b, out_a, out_b,
        isem_a, isem_b, wsem_a, wsem_b):
    wid = lax.axis_index("s") * _NC + lax.axis_index("c")
    iota16 = lax.iota(jnp.int32, 16)

    def fire_in(c, buf, sem):
      pltpu.async_copy(embt_hbm.at[:, pl.ds(c * _GRP, _GRP)], buf, sem)

    def drain_in(sem, buf):
      pltpu.make_async_copy(embt_hbm.at[:, pl.ds(0, _GRP)], buf, sem).wait()

    def fire_out(c, buf, sem):
      pltpu.async_copy(buf.at[pl.ds(0, _D)], out_hbm.at[pl.ds(c * 32, 32)],
                       sem)

    def drain_out(sem, buf):
      pltpu.make_async_copy(out_hbm.at[pl.ds(0, 32)], buf.at[pl.ds(0, _D)],
                            sem).wait()

    def transpose_blk(src, dst):
      @functools.partial(plsc.parallel_loop, 0, 256, unroll=8)
      def _t(t):
        a_vec = iota16 + lax.shift_right_logical(t, 5) * 16
        b_vec = (iota16 + (t & 31)) & 31
        v = plsc.load_gather(src, [b_vec, a_vec])
        f = a_vec * 32 + b_vec
        plsc.store_scatter(
            dst, [lax.shift_right_logical(f, 7), f & 127], v)

    fire_in(wid, in_a, isem_a)

    @pl.loop(0, _BLK_PER_W, step=2)
    def _blocks(j0):
      c0 = wid + j0 * _NW
      drain_in(isem_a, in_a)
      fire_in(c0 + _NW, in_b, isem_b)
      @pl.when(j0 >= 2)
      def _():
        drain_out(wsem_a, out_a)
      transpose_blk(in_a, out_a)
      fire_out(c0, out_a, wsem_a)
      drain_in(isem_b, in_b)
      @pl.when(j0 + 2 < _BLK_PER_W)
      def _():
        fire_in(c0 + 2 * _NW, in_a, isem_a)
      @pl.when(j0 >= 1)
      def _():
        drain_out(wsem_b, out_b)
      transpose_blk(in_b, out_b)
      fire_out(c0 + _NW, out_b, wsem_b)

    drain_out(wsem_a, out_a)
    drain_out(wsem_b, out_b)

    @pl.when(wid < 4)
    def _():
      c = _BLK_PER_W * _NW + wid
      pltpu.sync_copy(embt_hbm.at[:, pl.ds(c * _GRP, _GRP)], in_a)
      transpose_blk(in_a, out_a)
      pltpu.sync_copy(out_a.at[pl.ds(0, _D)], out_hbm.at[pl.ds(c * 32, 32)])

    @pl.when(wid == 4)
    def _():
      pltpu.sync_copy(tail_hbm, in_a.at[pl.ds(0, 16)])
      pltpu.sync_copy(in_a.at[pl.ds(0, 16)],
                      out_hbm.at[pl.ds(_NBLK * 32, 16)])

  return k


def _make_gather():
  mesh = plsc.VectorSubcoreMesh(core_axis_name="c", subcore_axis_name="s")

  @functools.partial(
      pl.kernel,
      out_type=jax.ShapeDtypeStruct((4, _B // _GRP, 8, _GRP), jnp.float32),
      mesh=mesh,
      scratch_types=[
          pltpu.VMEM((_NGRP, _GRP), jnp.int32),
          pltpu.VMEM((_CG * _GRP, _D), jnp.float32),
          pltpu.VMEM((_CG * _GRP, _D), jnp.float32),
          pltpu.VMEM((_D, _CG * _GRP), jnp.float32),
          pltpu.VMEM((_D, _CG * _GRP), jnp.float32),
          pltpu.SemaphoreType.DMA,
          pltpu.SemaphoreType.DMA,
          pltpu.SemaphoreType.DMA,
          pltpu.SemaphoreType.DMA,
      ],
      compiler_params=pltpu.CompilerParams(
          use_tc_tiling_on_sc=False, needs_layout_passes=False),
  )
  def k(src_hbm, tab_hbm, out_hbm, idx_v, rows_a, rows_b, oblk_a, oblk_b,
        gsem_a, gsem_b, osem_a, osem_b):
    wid = lax.axis_index("s") * _NC + lax.axis_index("c")
    pltpu.sync_copy(src_hbm.at[wid], idx_v)
    gbase = wid * _NGRP
    iota16 = lax.iota(jnp.int32, 16)

    def fire_gathers(c, rows, sem):
      for g in range(_CG):
        pltpu.async_copy(
            tab_hbm.at[idx_v.at[c * _CG + g]],
            rows.at[pl.ds(g * _GRP, _GRP)], sem)

    def drain_g(sem, rows):
      pltpu.make_async_copy(tab_hbm.at[pl.ds(0, _CG * _GRP)], rows, sem).wait()

    def drain_o(sem, oblk):
      for r in range(4):
        for g in range(_CG):
          pltpu.make_async_copy(
              out_hbm.at[0, 0],
              oblk.at[pl.ds(8 * r, 8), pl.ds(g * _GRP, _GRP)], sem).wait()

    def transpose_chunk(rows, oblk):
      @functools.partial(plsc.parallel_loop, 0, _CG * 256, unroll=8)
      def _t(t):
        col_vec = iota16 + lax.shift_right_logical(t, 5) * 16
        d_vec = (iota16 + (t & 31)) & 31
        v = plsc.load_gather(rows, [col_vec, d_vec])
        plsc.store_scatter(oblk, [d_vec, col_vec], v)

    def fire_out(c, oblk, sem):
      for r in range(4):
        for g in range(_CG):
          pltpu.async_copy(
              oblk.at[pl.ds(8 * r, 8), pl.ds(g * _GRP, _GRP)],
              out_hbm.at[r, gbase + c * _CG + g], sem)

    fire_gathers(0, rows_a, gsem_a)

    @pl.loop(0, _NCHUNK, step=2)
    def _chunks(c0):
      drain_g(gsem_a, rows_a)
      fire_gathers(c0 + 1, rows_b, gsem_b)
      @pl.when(c0 >= 2)
      def _():
        drain_o(osem_a, oblk_a)
      transpose_chunk(rows_a, oblk_a)
      fire_out(c0, oblk_a, osem_a)
      drain_g(gsem_b, rows_b)
      @pl.when(c0 + 2 < _NCHUNK)
      def _():
        fire_gathers(c0 + 2, rows_a, gsem_a)
      @pl.when(c0 >= 1)
      def _():
        drain_o(osem_b, oblk_b)
      transpose_chunk(rows_b, oblk_b)
      fire_out(c0 + 1, oblk_b, osem_b)

    drain_o(osem_a, oblk_a)
    drain_o(osem_b, oblk_b)

  return k


_transpose = _make_transpose()
_gather = _make_gather()


def kernel(inputs, embeddings):
  embt = jnp.transpose(embeddings)
  tail_rm = jnp.reshape(embeddings[_NBLK * _GRP:, :], (16, 128))
  table_pk = _transpose(embt, tail_rm)
  tab = jnp.reshape(table_pk, (_V, _D))
  src = jnp.reshape(inputs.astype(jnp.int32), (_NW, _NGRP, _GRP))
  out4 = _gather(src, tab)
  return jnp.reshape(jnp.transpose(out4, (1, 3, 0, 2)), (_B, _D))
